# baseline (speedup 1.0000x reference)
import functools
import jax
import jax.numpy as jnp
from jax import lax
from jax.experimental import pallas as pl
from jax.experimental.pallas import tpu as pltpu
from jax.experimental.pallas import tpu_sc as plsc

BLK = 128
TI = 2048
_NC, _NS = 2, 16
_NW = _NC * _NS


def _sc_dispatch(x, pos, cap):
    t, h = x.shape
    na = 2 * t
    per_w = na // _NW
    mesh = plsc.VectorSubcoreMesh(core_axis_name="c", subcore_axis_name="s")

    @functools.partial(
        pl.kernel,
        out_type=jax.ShapeDtypeStruct((cap, h), jnp.float32),
        mesh=mesh,
        scratch_types=[
            pltpu.VMEM((per_w,), jnp.int32),
            pltpu.VMEM((per_w, h), jnp.float32),
            pltpu.SemaphoreType.DMA,
        ],
    )
    def k(x_hbm, pos_hbm, xg_hbm, pos_v, rows_v, sem):
        wid = lax.axis_index("s") * _NC + lax.axis_index("c")
        a_base = wid * per_w
        tok_base = lax.rem(a_base, t)
        pltpu.sync_copy(pos_hbm.at[pl.ds(a_base, per_w)], pos_v)
        pltpu.sync_copy(x_hbm.at[pl.ds(tok_base, per_w)], rows_v)
        pltpu.async_copy(rows_v, xg_hbm.at[pos_v], sem).wait()

    return k(x, pos)


def _sc_combine(y, p1, p2, w1, w2):
    t = p1.shape[0]
    h = y.shape[1]
    per_w = t // _NW
    ncol = h // 16
    mesh = plsc.VectorSubcoreMesh(core_axis_name="c", subcore_axis_name="s")

    @functools.partial(
        pl.kernel,
        out_type=jax.ShapeDtypeStruct((t, h), jnp.float32),
        mesh=mesh,
        scratch_types=[
            pltpu.VMEM((per_w,), jnp.int32),
            pltpu.VMEM((per_w,), jnp.int32),
            pltpu.VMEM((per_w,), jnp.float32),
            pltpu.VMEM((per_w,), jnp.float32),
            pltpu.VMEM((per_w, h), jnp.float32),
            pltpu.VMEM((per_w, h), jnp.float32),
            pltpu.SemaphoreType.DMA,
        ],
        compiler_params=pltpu.CompilerParams(needs_layout_passes=False),
    )
    def k(y_hbm, p1_hbm, p2_hbm, w1_hbm, w2_hbm, out_hbm,
          i1_v, i2_v, a1_v, a2_v, r1_v, r2_v, sem):
        wid = lax.axis_index("s") * _NC + lax.axis_index("c")
        base = wid * per_w
        pltpu.sync_copy(p1_hbm.at[pl.ds(base, per_w)], i1_v)
        pltpu.sync_copy(p2_hbm.at[pl.ds(base, per_w)], i2_v)
        pltpu.sync_copy(w1_hbm.at[pl.ds(base, per_w)], a1_v)
        pltpu.sync_copy(w2_hbm.at[pl.ds(base, per_w)], a2_v)
        c1 = pltpu.async_copy(y_hbm.at[i1_v], r1_v, sem)
        c2 = pltpu.async_copy(y_hbm.at[i2_v], r2_v, sem)
        c1.wait()
        c2.wait()

        def body(tok, carry):
            splat = jnp.full((16,), tok, jnp.int32)
            s1 = plsc.load_gather(a1_v, [splat])
            s2 = plsc.load_gather(a2_v, [splat])
            for c in range(ncol):
                sl = pl.ds(c * 16, 16)
                r1_v[tok, sl] = s1 * r1_v[tok, sl] + s2 * r2_v[tok, sl]
            return carry

        lax.fori_loop(0, per_w, body, 0)
        pltpu.sync_copy(r1_v, out_hbm.at[pl.ds(base, per_w)])

    return k(y, p1, p2, w1, w2)


def _route_kernel(x_ref, gw_ref, logits_ref, wcol_ref, pos_ref, aux_ref):
    x = x_ref[...]
    gw = gw_ref[...]
    l = jnp.dot(x, gw, preferred_element_type=jnp.float32)
    logits_ref[...] = l
    tt = l.shape[0]
    lane = jax.lax.broadcasted_iota(jnp.int32, l.shape, 1)
    lanef = lane.astype(jnp.float32)
    valid = lane < 8
    neg = jnp.float32(-jnp.inf)
    lm = jnp.where(valid, l, neg)
    m1 = jnp.max(lm, axis=1, keepdims=True)
    i1 = jnp.min(jnp.where(lm == m1, lane, 128), axis=1, keepdims=True)
    s = jnp.sum(jnp.where(valid, jnp.exp(lm - m1), 0.0), axis=1, keepdims=True)
    lm2 = jnp.where(lane == i1, neg, lm)
    m2 = jnp.max(lm2, axis=1, keepdims=True)
    i2 = jnp.min(jnp.where(lm2 == m2, lane, 128), axis=1, keepdims=True)
    w1 = 1.0 / s
    w2 = jnp.exp(m2 - m1) / s
    wcol_ref[...] = (jnp.where(lane == 0, w1, 0.0)
                     + jnp.where(lane == 1, w2, 0.0))
    i1f = i1.astype(jnp.float32)
    i2f = i2.astype(jnp.float32)
    oh = (jnp.where(lanef == i1f, 1.0, 0.0)
          + jnp.where(lanef == i2f + 8.0, 1.0, 0.0))
    rr = lax.broadcasted_iota(jnp.int32, (tt, tt), 0)
    cc = lax.broadcasted_iota(jnp.int32, (tt, tt), 1)
    tril = jnp.where(rr >= cc, 1.0, 0.0)
    csum = jnp.dot(tril, oh, preferred_element_type=jnp.float32)
    last = csum[tt - 1:tt, :]
    lr = lax.broadcasted_iota(jnp.int32, (128, 128), 0)
    lc = lax.broadcasted_iota(jnp.int32, (128, 128), 1)
    fold = jnp.where((lr == lc) | (lr == lc + 8), 1.0, 0.0)
    fold = fold * jnp.where(lc < 8, 1.0, 0.0)
    counts = jnp.dot(last, fold, preferred_element_type=jnp.float32)
    padded = jnp.ceil(counts / 128.0) * 128.0
    strict = jnp.where(lr < lc, 1.0, 0.0)
    offs = jnp.dot(padded, strict, preferred_element_type=jnp.float32)
    rank1 = jnp.sum(jnp.where(lanef == i1f, csum, 0.0), axis=1,
                    keepdims=True) - 1.0
    cnt1_at_i2 = jnp.sum(jnp.where(lanef == i2f, last, 0.0), axis=1,
                         keepdims=True)
    rank2 = cnt1_at_i2 + jnp.sum(
        jnp.where(lanef == i2f + 8.0, csum, 0.0), axis=1, keepdims=True) - 1.0
    off1 = jnp.sum(jnp.where(lanef == i1f, offs, 0.0), axis=1, keepdims=True)
    off2 = jnp.sum(jnp.where(lanef == i2f, offs, 0.0), axis=1, keepdims=True)
    pos = (jnp.where(lane == 0, off1 + rank1, 0.0)
           + jnp.where(lane == 1, off2 + rank2, 0.0))
    pos_ref[...] = pos.astype(jnp.int32)
    blko_col = jnp.transpose(offs / 128.0)
    ge = jnp.where((lc.astype(jnp.float32) >= blko_col)
                   & (lr < 8), 1.0, 0.0)
    be_row = jnp.dot(jnp.full((1, 128), 1.0, jnp.float32),
                     ge, preferred_element_type=jnp.float32) - 1.0
    be_row = jnp.clip(be_row, 0.0, 7.0)
    nbu = jnp.sum(jnp.where(lane[0:1, :] == 7, offs + padded, 0.0),
                  axis=1, keepdims=True) / 128.0
    row8 = lax.broadcasted_iota(jnp.int32, (8, 128), 0)
    aux = (jnp.where(row8 == 0, be_row, 0.0)
           + jnp.where(row8 == 1, nbu, 0.0))
    aux_ref[...] = aux.astype(jnp.int32)


def _ffn_kernel(be_ref, nbu_ref, xg_ref, win_ref, wv_ref, wout_ref, y_ref):
    i = pl.program_id(0)
    bb = pl.program_id(1)
    ni = pl.num_programs(0)
    valid = bb < nbu_ref[0]

    @pl.when(valid)
    def _():
        xb = xg_ref[...]
        dotp = functools.partial(
            jax.lax.dot_general,
            dimension_numbers=(((1,), (0,)), ((), ())),
            precision=lax.Precision.DEFAULT,
            preferred_element_type=jnp.float32)
        up = dotp(xb, win_ref[0])
        v = dotp(xb, wv_ref[0])
        hg = jax.nn.gelu(up) * v
        part = dotp(hg, wout_ref[0])

        rows = pl.ds(bb * BLK, BLK)
        if ni == 1:
            y_ref[rows, :] = part
        else:
            @pl.when(i == 0)
            def _():
                y_ref[rows, :] = part

            @pl.when(i > 0)
            def _():
                y_ref[rows, :] += part


def kernel(hidden, gate_w, w_in, w_v, w_out):
    b, s, h = hidden.shape
    e = gate_w.shape[1]
    ii = w_in.shape[2]
    t = b * s
    x = hidden.reshape(t, h)

    cap = 2 * t + e * BLK
    nb = cap // BLK
    gw_pad = jnp.zeros((h, 128), jnp.float32).at[:, :e].set(gate_w)
    logits_pad, wcol, pos2d, aux = pl.pallas_call(
        _route_kernel,
        out_shape=[
            jax.ShapeDtypeStruct((t, 128), jnp.float32),
            jax.ShapeDtypeStruct((t, 128), jnp.float32),
            jax.ShapeDtypeStruct((t, 128), jnp.int32),
            jax.ShapeDtypeStruct((8, 128), jnp.int32),
        ],
    )(x, gw_pad)
    router_logits = logits_pad[:, :e]
    w1 = wcol[:, 0]
    w2 = wcol[:, 1]
    p1, p2 = pos2d[:, 0], pos2d[:, 1]
    pos = jnp.concatenate([p1, p2])
    block_expert = aux[0, :nb]
    nb_used = aux[1, :1]

    xg = _sc_dispatch(x, pos, cap)

    ni = ii // TI
    grid_spec = pltpu.PrefetchScalarGridSpec(
        num_scalar_prefetch=2,
        grid=(ni, nb),
        in_specs=[
            pl.BlockSpec((BLK, h), lambda i, bb, be, nbu: (bb, 0)),
            pl.BlockSpec((1, h, TI), lambda i, bb, be, nbu: (be[bb], 0, i)),
            pl.BlockSpec((1, h, TI), lambda i, bb, be, nbu: (be[bb], 0, i)),
            pl.BlockSpec((1, TI, h), lambda i, bb, be, nbu: (be[bb], i, 0)),
        ],
        out_specs=pl.BlockSpec((cap, h), lambda i, bb, be, nbu: (0, 0)),
    )
    y = pl.pallas_call(
        _ffn_kernel,
        grid_spec=grid_spec,
        out_shape=jax.ShapeDtypeStruct((cap, h), jnp.float32),
        compiler_params=pltpu.CompilerParams(
            dimension_semantics=("arbitrary", "arbitrary"),
            vmem_limit_bytes=120 * 1024 * 1024),
    )(block_expert, nb_used, xg, w_in, w_v, w_out)

    out = _sc_combine(y, p1, p2, w1, w2)
    return out.reshape(b, s, h), router_logits


# speedup vs baseline: 1.0196x; 1.0196x over previous
import functools
import jax
import jax.numpy as jnp
from jax import lax
from jax.experimental import pallas as pl
from jax.experimental.pallas import tpu as pltpu
from jax.experimental.pallas import tpu_sc as plsc

BLK = 128
TI = 2048
_NC, _NS = 2, 16
_NW = _NC * _NS


def _sc_dispatch(x, pos, cap):
    t, h = x.shape
    na = 2 * t
    per_w = na // _NW
    mesh = plsc.VectorSubcoreMesh(core_axis_name="c", subcore_axis_name="s")

    @functools.partial(
        pl.kernel,
        out_type=jax.ShapeDtypeStruct((cap, h), jnp.float32),
        mesh=mesh,
        scratch_types=[
            pltpu.VMEM((per_w,), jnp.int32),
            pltpu.VMEM((per_w, h), jnp.float32),
            pltpu.SemaphoreType.DMA,
        ],
    )
    def k(x_hbm, pos_hbm, xg_hbm, pos_v, rows_v, sem):
        wid = lax.axis_index("s") * _NC + lax.axis_index("c")
        a_base = wid * per_w
        tok_base = lax.rem(a_base, t)
        pltpu.sync_copy(pos_hbm.at[pl.ds(a_base, per_w)], pos_v)
        pltpu.sync_copy(x_hbm.at[pl.ds(tok_base, per_w)], rows_v)
        pltpu.async_copy(rows_v, xg_hbm.at[pos_v], sem).wait()

    return k(x, pos)


def _sc_combine(y, p1, p2, w1, w2):
    t = p1.shape[0]
    h = y.shape[1]
    per_w = t // _NW
    ch = per_w // 2
    ncol = h // 16
    mesh = plsc.VectorSubcoreMesh(core_axis_name="c", subcore_axis_name="s")

    @functools.partial(
        pl.kernel,
        out_type=jax.ShapeDtypeStruct((t, h), jnp.float32),
        mesh=mesh,
        scratch_types=[
            pltpu.VMEM((per_w,), jnp.int32),
            pltpu.VMEM((per_w,), jnp.int32),
            pltpu.VMEM((per_w,), jnp.float32),
            pltpu.VMEM((per_w,), jnp.float32),
            [pltpu.VMEM((ch, h), jnp.float32) for _ in range(2)],
            [pltpu.VMEM((ch, h), jnp.float32) for _ in range(2)],
            pltpu.SemaphoreType.DMA,
            pltpu.SemaphoreType.DMA,
        ],
        compiler_params=pltpu.CompilerParams(needs_layout_passes=False),
    )
    def k(y_hbm, p1_hbm, p2_hbm, w1_hbm, w2_hbm, out_hbm,
          i1_v, i2_v, a1_v, a2_v, r1, r2, gsem, osem):
        wid = lax.axis_index("s") * _NC + lax.axis_index("c")
        base = wid * per_w
        pltpu.sync_copy(p1_hbm.at[pl.ds(base, per_w)], i1_v)
        pltpu.sync_copy(p2_hbm.at[pl.ds(base, per_w)], i2_v)
        pltpu.sync_copy(w1_hbm.at[pl.ds(base, per_w)], a1_v)
        pltpu.sync_copy(w2_hbm.at[pl.ds(base, per_w)], a2_v)
        copies = []
        for half in range(2):
            sl = pl.ds(half * ch, ch)
            copies.append(pltpu.async_copy(y_hbm.at[i1_v.at[sl]], r1[half],
                                           gsem))
            copies.append(pltpu.async_copy(y_hbm.at[i2_v.at[sl]], r2[half],
                                           gsem))
        outs = []
        for half in range(2):
            copies[2 * half].wait()
            copies[2 * half + 1].wait()

            def body(tok, carry, half=half):
                splat = jnp.full((16,), half * ch + tok, jnp.int32)
                s1 = plsc.load_gather(a1_v, [splat])
                s2 = plsc.load_gather(a2_v, [splat])
                for c in range(ncol):
                    cs = pl.ds(c * 16, 16)
                    r1[half][tok, cs] = (s1 * r1[half][tok, cs]
                                         + s2 * r2[half][tok, cs])
                return carry

            lax.fori_loop(0, ch, body, 0)
            outs.append(pltpu.async_copy(
                r1[half], out_hbm.at[pl.ds(base + half * ch, ch)], osem))
        for o in outs:
            o.wait()

    return k(y, p1, p2, w1, w2)


def _route_kernel(x_ref, gw_ref, logits_ref, wcol_ref, pos_ref, aux_ref):
    x = x_ref[...]
    gw = gw_ref[...]
    l = jnp.dot(x, gw, preferred_element_type=jnp.float32)
    logits_ref[...] = l
    tt = l.shape[0]
    lane = jax.lax.broadcasted_iota(jnp.int32, l.shape, 1)
    lanef = lane.astype(jnp.float32)
    valid = lane < 8
    neg = jnp.float32(-jnp.inf)
    lm = jnp.where(valid, l, neg)
    m1 = jnp.max(lm, axis=1, keepdims=True)
    i1 = jnp.min(jnp.where(lm == m1, lane, 128), axis=1, keepdims=True)
    s = jnp.sum(jnp.where(valid, jnp.exp(lm - m1), 0.0), axis=1, keepdims=True)
    lm2 = jnp.where(lane == i1, neg, lm)
    m2 = jnp.max(lm2, axis=1, keepdims=True)
    i2 = jnp.min(jnp.where(lm2 == m2, lane, 128), axis=1, keepdims=True)
    w1 = 1.0 / s
    w2 = jnp.exp(m2 - m1) / s
    wcol_ref[...] = (jnp.where(lane == 0, w1, 0.0)
                     + jnp.where(lane == 1, w2, 0.0))
    i1f = i1.astype(jnp.float32)
    i2f = i2.astype(jnp.float32)
    oh = (jnp.where(lanef == i1f, 1.0, 0.0)
          + jnp.where(lanef == i2f + 8.0, 1.0, 0.0))
    rr = lax.broadcasted_iota(jnp.int32, (tt, tt), 0)
    cc = lax.broadcasted_iota(jnp.int32, (tt, tt), 1)
    tril = jnp.where(rr >= cc, 1.0, 0.0)
    csum = jnp.dot(tril, oh, preferred_element_type=jnp.float32)
    last = csum[tt - 1:tt, :]
    lr = lax.broadcasted_iota(jnp.int32, (128, 128), 0)
    lc = lax.broadcasted_iota(jnp.int32, (128, 128), 1)
    fold = jnp.where((lr == lc) | (lr == lc + 8), 1.0, 0.0)
    fold = fold * jnp.where(lc < 8, 1.0, 0.0)
    counts = jnp.dot(last, fold, preferred_element_type=jnp.float32)
    padded = jnp.ceil(counts / 128.0) * 128.0
    strict = jnp.where(lr < lc, 1.0, 0.0)
    offs = jnp.dot(padded, strict, preferred_element_type=jnp.float32)
    rank1 = jnp.sum(jnp.where(lanef == i1f, csum, 0.0), axis=1,
                    keepdims=True) - 1.0
    cnt1_at_i2 = jnp.sum(jnp.where(lanef == i2f, last, 0.0), axis=1,
                         keepdims=True)
    rank2 = cnt1_at_i2 + jnp.sum(
        jnp.where(lanef == i2f + 8.0, csum, 0.0), axis=1, keepdims=True) - 1.0
    off1 = jnp.sum(jnp.where(lanef == i1f, offs, 0.0), axis=1, keepdims=True)
    off2 = jnp.sum(jnp.where(lanef == i2f, offs, 0.0), axis=1, keepdims=True)
    pos = (jnp.where(lane == 0, off1 + rank1, 0.0)
           + jnp.where(lane == 1, off2 + rank2, 0.0))
    pos_ref[...] = pos.astype(jnp.int32)
    blko_col = jnp.transpose(offs / 128.0)
    ge = jnp.where((lc.astype(jnp.float32) >= blko_col)
                   & (lr < 8), 1.0, 0.0)
    be_row = jnp.dot(jnp.full((1, 128), 1.0, jnp.float32),
                     ge, preferred_element_type=jnp.float32) - 1.0
    be_row = jnp.clip(be_row, 0.0, 7.0)
    nbu = jnp.sum(jnp.where(lane[0:1, :] == 7, offs + padded, 0.0),
                  axis=1, keepdims=True) / 128.0
    row8 = lax.broadcasted_iota(jnp.int32, (8, 128), 0)
    aux = (jnp.where(row8 == 0, be_row, 0.0)
           + jnp.where(row8 == 1, nbu, 0.0))
    aux_ref[...] = aux.astype(jnp.int32)


def _ffn_kernel(be_ref, nbu_ref, xg_ref, win_ref, wv_ref, wout_ref, y_ref):
    i = pl.program_id(0)
    bb = pl.program_id(1)
    ni = pl.num_programs(0)
    valid = bb < nbu_ref[0]

    @pl.when(valid)
    def _():
        xb = xg_ref[...].astype(jnp.bfloat16)
        dotp = functools.partial(
            jax.lax.dot_general,
            dimension_numbers=(((1,), (0,)), ((), ())),
            precision=lax.Precision.DEFAULT,
            preferred_element_type=jnp.float32)
        up = dotp(xb, win_ref[0])
        v = dotp(xb, wv_ref[0])
        hg = jax.nn.gelu(up) * v
        part = dotp(hg, wout_ref[0])

        rows = pl.ds(bb * BLK, BLK)
        if ni == 1:
            y_ref[rows, :] = part
        else:
            @pl.when(i == 0)
            def _():
                y_ref[rows, :] = part

            @pl.when(i > 0)
            def _():
                y_ref[rows, :] += part


def kernel(hidden, gate_w, w_in, w_v, w_out):
    b, s, h = hidden.shape
    e = gate_w.shape[1]
    ii = w_in.shape[2]
    t = b * s
    x = hidden.reshape(t, h)

    cap = 2 * t + e * BLK
    nb = cap // BLK
    gw_pad = jnp.zeros((h, 128), jnp.float32).at[:, :e].set(gate_w)
    logits_pad, wcol, pos2d, aux = pl.pallas_call(
        _route_kernel,
        out_shape=[
            jax.ShapeDtypeStruct((t, 128), jnp.float32),
            jax.ShapeDtypeStruct((t, 128), jnp.float32),
            jax.ShapeDtypeStruct((t, 128), jnp.int32),
            jax.ShapeDtypeStruct((8, 128), jnp.int32),
        ],
    )(x, gw_pad)
    router_logits = logits_pad[:, :e]
    w1 = wcol[:, 0]
    w2 = wcol[:, 1]
    p1, p2 = pos2d[:, 0], pos2d[:, 1]
    pos = jnp.concatenate([p1, p2])
    block_expert = aux[0, :nb]
    nb_used = aux[1, :1]

    xg = _sc_dispatch(x, pos, cap)

    ni = ii // TI
    grid_spec = pltpu.PrefetchScalarGridSpec(
        num_scalar_prefetch=2,
        grid=(ni, nb),
        in_specs=[
            pl.BlockSpec((BLK, h), lambda i, bb, be, nbu: (bb, 0)),
            pl.BlockSpec((1, h, TI), lambda i, bb, be, nbu: (be[bb], 0, i)),
            pl.BlockSpec((1, h, TI), lambda i, bb, be, nbu: (be[bb], 0, i)),
            pl.BlockSpec((1, TI, h), lambda i, bb, be, nbu: (be[bb], i, 0)),
        ],
        out_specs=pl.BlockSpec((cap, h), lambda i, bb, be, nbu: (0, 0)),
    )
    y = pl.pallas_call(
        _ffn_kernel,
        grid_spec=grid_spec,
        out_shape=jax.ShapeDtypeStruct((cap, h), jnp.float32),
        compiler_params=pltpu.CompilerParams(
            dimension_semantics=("arbitrary", "arbitrary"),
            vmem_limit_bytes=120 * 1024 * 1024),
    )(block_expert, nb_used, xg, w_in, w_v, w_out)

    out = _sc_combine(y, p1, p2, w1, w2)
    return out.reshape(b, s, h), router_logits
